# baseline (speedup 1.0000x reference)
import functools

import jax
import jax.numpy as jnp
from jax import lax
from jax.experimental import pallas as pl
import jax.experimental.pallas.tpu as pltpu
from jax.experimental.pallas import tpu_sc as plsc

SEQ_LEN = 2048
D_MODEL = 1024
N_REP = 4
NUM_CORES = 2
HALF = SEQ_LEN // NUM_CORES

_mesh = plsc.ScalarSubcoreMesh(axis_name="c", num_cores=NUM_CORES)


@functools.partial(
    pl.kernel,
    mesh=_mesh,
    out_type=jax.ShapeDtypeStruct((SEQ_LEN, N_REP, D_MODEL), jnp.float32),
    scratch_types=[pltpu.SemaphoreType.DMA],
)
def _sc_body(pe_hbm, out_hbm, sem):
    base = lax.axis_index("c") * HALF
    copies = [
        pltpu.async_copy(
            pe_hbm.at[pl.ds(base, HALF)],
            out_hbm.at[pl.ds(base, HALF), n],
            sem,
        )
        for n in range(N_REP)
    ]
    for c in copies:
        c.wait()


def kernel(z, pos_embed):
    del z
    return _sc_body(pos_embed)


# speedup vs baseline: 31.1988x; 31.1988x over previous
import functools

import jax
import jax.numpy as jnp
from jax import lax
from jax.experimental import pallas as pl
import jax.experimental.pallas.tpu as pltpu
from jax.experimental.pallas import tpu_sc as plsc

SEQ_LEN = 2048
D_MODEL = 1024
N_REP = 4
NUM_CORES = 2
NUM_SUBCORES = 16
NUM_WORKERS = NUM_CORES * NUM_SUBCORES
CHUNK = SEQ_LEN // NUM_WORKERS
SUB = CHUNK // 2

_mesh = plsc.VectorSubcoreMesh(core_axis_name="c", subcore_axis_name="s")


@functools.partial(
    pl.kernel,
    mesh=_mesh,
    out_type=jax.ShapeDtypeStruct((SEQ_LEN, N_REP, D_MODEL), jnp.float32),
    scratch_types=[
        pltpu.VMEM((SUB, D_MODEL), jnp.float32),
        pltpu.VMEM((SUB, D_MODEL), jnp.float32),
        pltpu.SemaphoreType.DMA,
        pltpu.SemaphoreType.DMA,
        pltpu.SemaphoreType.DMA,
    ],
)
def _sc_body(pe_hbm, out_hbm, buf0, buf1, rsem0, rsem1, wsem):
    wid = lax.axis_index("s") * NUM_CORES + lax.axis_index("c")
    base = wid * CHUNK
    r0 = pltpu.async_copy(pe_hbm.at[pl.ds(base, SUB)], buf0, rsem0)
    r1 = pltpu.async_copy(pe_hbm.at[pl.ds(base + SUB, SUB)], buf1, rsem1)
    r0.wait()
    writes = [
        pltpu.async_copy(buf0, out_hbm.at[pl.ds(base, SUB), n], wsem)
        for n in range(N_REP)
    ]
    r1.wait()
    writes += [
        pltpu.async_copy(buf1, out_hbm.at[pl.ds(base + SUB, SUB), n], wsem)
        for n in range(N_REP)
    ]
    for c in writes:
        c.wait()


def kernel(z, pos_embed):
    del z
    return _sc_body(pos_embed)
